# baseline (speedup 1.0000x reference)
import jax
import jax.numpy as jnp
from jax.experimental import pallas as pl
from jax.experimental.pallas import tpu as pltpu

SEQ = 24
INV = 16
F = SEQ * INV


def _fold_kernel(fei_ref, wlt_ref, wrt_ref, bl_ref, w1p_ref, w2p_ref,
                 fcwt_ref, fcb_ref, m_ref, c_ref):
    f32 = jnp.float32
    ne = fei_ref.shape[1]
    si = fei_ref[0:1, :]
    di = fei_ref[1:2, :]
    node_iota = jax.lax.broadcasted_iota(jnp.int32, (INV, ne), 0)
    S = (node_iota == si).astype(f32)
    D = (node_iota == di).astype(f32)
    acnt = jax.lax.dot_general(D, S, (((1,), (1,)), ((), ())),
                               preferred_element_type=f32)
    cnt = jnp.sum(acnt, axis=1, keepdims=True)
    A = acnt / jnp.maximum(cnt, 1.0)

    fc1t = fcwt_ref[0:INV, :]
    fc2t = fcwt_ref[INV:2 * INV, :]
    Q1 = jax.lax.dot_general(A, fc1t, (((0,), (0,)), ((), ())),
                             preferred_element_type=f32)
    Q2 = fc1t

    ri = jax.lax.broadcasted_iota(jnp.int32, (F, SEQ), 0) // INV
    sj = jax.lax.broadcasted_iota(jnp.int32, (F, SEQ), 1)
    R = (ri == sj).astype(f32)
    ti = jax.lax.broadcasted_iota(jnp.int32, (F, INV), 0) % INV
    tj = jax.lax.broadcasted_iota(jnp.int32, (F, INV), 1)
    T = (ti == tj).astype(f32)

    def expand_p(P):
        RP = jnp.dot(R, P, preferred_element_type=f32)
        return jax.lax.dot_general(RP, R, (((1,), (1,)), ((), ())),
                                   preferred_element_type=f32)

    def tile_q(Q):
        TQ = jnp.dot(T, Q, preferred_element_type=f32)
        return jax.lax.dot_general(TQ, T, (((1,), (1,)), ((), ())),
                                   preferred_element_type=f32)

    rr = jax.lax.broadcasted_iota(jnp.int32, (F, F), 0)
    cc = jax.lax.broadcasted_iota(jnp.int32, (F, F), 1)
    M = (rr == cc).astype(f32)
    M = M + expand_p(wlt_ref[...]) * tile_q(Q1)
    M = M + expand_p(wrt_ref[...]) * tile_q(Q2)

    rf = rr // INV
    cf = cc // INV
    for d in range(-2, 3):
        V = jnp.dot(w2p_ref[d + 2], fc2t, preferred_element_type=f32)
        if abs(d) <= 1:
            V = V + jnp.dot(w1p_ref[d + 1], fc2t, preferred_element_type=f32)
        band = (rf - cf == d).astype(f32)
        M = M + band * tile_q(V)
    m_ref[...] = M

    rs1 = jnp.sum(fc1t, axis=0, keepdims=True)
    bl_big = jax.lax.dot_general(bl_ref[...], R, (((1,), (1,)), ((), ())),
                                 preferred_element_type=f32)
    rs1_big = jax.lax.dot_general(rs1, T, (((1,), (1,)), ((), ())),
                                  preferred_element_type=f32)
    fcb_big = jax.lax.dot_general(fcb_ref[...], T, (((1,), (1,)), ((), ())),
                                  preferred_element_type=f32)
    c_ref[...] = bl_big * rs1_big + fcb_big


def _apply_kernel(x_ref, m_ref, c_ref, buf_ref, o_ref):
    del buf_ref
    y = jnp.dot(x_ref[...], m_ref[...],
                preferred_element_type=jnp.float32) + c_ref[...]
    o_ref[...] = y.astype(jnp.bfloat16)


def kernel(src, graph_edge_index, feature_edge_index, Wl, bl, Wr,
           conv1_w, conv2_w, fc_w, fc_b):
    bz, cap, seq, inv = src.shape
    B = bz * cap
    fei = jnp.pad(feature_edge_index, ((0, 6), (0, 0)))
    w1p = jnp.transpose(conv1_w, (2, 1, 0))
    w2p = jnp.transpose(conv2_w, (2, 1, 0))

    M, c = pl.pallas_call(
        _fold_kernel,
        out_shape=(jax.ShapeDtypeStruct((F, F), jnp.float32),
                   jax.ShapeDtypeStruct((1, F), jnp.float32)),
    )(fei, Wl.T, Wr.T, bl.reshape(1, seq), w1p, w2p, fc_w.T,
      fc_b.reshape(1, inv))

    Mb = M.astype(jnp.bfloat16)
    K = 4
    bz_k = bz // K
    Bk = bz_k * cap
    Bb = 4096
    buf = jnp.zeros((B, F), jnp.bfloat16)
    for k in range(K):
        xk = src[k * bz_k:(k + 1) * bz_k].astype(jnp.bfloat16).reshape(Bk, F)
        off = k * (Bk // Bb)
        buf = pl.pallas_call(
            _apply_kernel,
            grid=(Bk // Bb,),
            in_specs=[pl.BlockSpec((Bb, F), lambda i: (i, 0)),
                      pl.BlockSpec((F, F), lambda i: (0, 0)),
                      pl.BlockSpec((1, F), lambda i: (0, 0)),
                      pl.BlockSpec(memory_space=pl.ANY)],
            out_specs=pl.BlockSpec((Bb, F), lambda i, o=off: (o + i, 0)),
            out_shape=jax.ShapeDtypeStruct((B, F), jnp.bfloat16),
            input_output_aliases={3: 0},
        )(xk, Mb, c, buf)
    return buf.reshape(bz, cap, seq, inv).astype(jnp.float32)


# speedup vs baseline: 1.1838x; 1.1838x over previous
import jax
import jax.numpy as jnp
from jax.experimental import pallas as pl
from jax.experimental.pallas import tpu as pltpu

SEQ = 24
INV = 16
F = SEQ * INV


def _fold_kernel(fei_ref, wlt_ref, wrt_ref, bl_ref, w1p_ref, w2p_ref,
                 fcwt_ref, fcb_ref, m_ref, c_ref):
    f32 = jnp.float32
    ne = fei_ref.shape[1]
    si = fei_ref[0:1, :]
    di = fei_ref[1:2, :]
    node_iota = jax.lax.broadcasted_iota(jnp.int32, (INV, ne), 0)
    S = (node_iota == si).astype(f32)
    D = (node_iota == di).astype(f32)
    acnt = jax.lax.dot_general(D, S, (((1,), (1,)), ((), ())),
                               preferred_element_type=f32)
    cnt = jnp.sum(acnt, axis=1, keepdims=True)
    A = acnt / jnp.maximum(cnt, 1.0)

    fc1t = fcwt_ref[0:INV, :]
    fc2t = fcwt_ref[INV:2 * INV, :]
    Q1 = jax.lax.dot_general(A, fc1t, (((0,), (0,)), ((), ())),
                             preferred_element_type=f32)
    Q2 = fc1t

    ri = jax.lax.broadcasted_iota(jnp.int32, (F, SEQ), 0) // INV
    sj = jax.lax.broadcasted_iota(jnp.int32, (F, SEQ), 1)
    R = (ri == sj).astype(f32)
    ti = jax.lax.broadcasted_iota(jnp.int32, (F, INV), 0) % INV
    tj = jax.lax.broadcasted_iota(jnp.int32, (F, INV), 1)
    T = (ti == tj).astype(f32)

    def expand_p(P):
        RP = jnp.dot(R, P, preferred_element_type=f32)
        return jax.lax.dot_general(RP, R, (((1,), (1,)), ((), ())),
                                   preferred_element_type=f32)

    def tile_q(Q):
        TQ = jnp.dot(T, Q, preferred_element_type=f32)
        return jax.lax.dot_general(TQ, T, (((1,), (1,)), ((), ())),
                                   preferred_element_type=f32)

    rr = jax.lax.broadcasted_iota(jnp.int32, (F, F), 0)
    cc = jax.lax.broadcasted_iota(jnp.int32, (F, F), 1)
    M = (rr == cc).astype(f32)
    M = M + expand_p(wlt_ref[...]) * tile_q(Q1)
    M = M + expand_p(wrt_ref[...]) * tile_q(Q2)

    rf = rr // INV
    cf = cc // INV
    for d in range(-2, 3):
        V = jnp.dot(w2p_ref[d + 2], fc2t, preferred_element_type=f32)
        if abs(d) <= 1:
            V = V + jnp.dot(w1p_ref[d + 1], fc2t, preferred_element_type=f32)
        band = (rf - cf == d).astype(f32)
        M = M + band * tile_q(V)
    m_ref[...] = M

    rs1 = jnp.sum(fc1t, axis=0, keepdims=True)
    bl_big = jax.lax.dot_general(bl_ref[...], R, (((1,), (1,)), ((), ())),
                                 preferred_element_type=f32)
    rs1_big = jax.lax.dot_general(rs1, T, (((1,), (1,)), ((), ())),
                                  preferred_element_type=f32)
    fcb_big = jax.lax.dot_general(fcb_ref[...], T, (((1,), (1,)), ((), ())),
                                  preferred_element_type=f32)
    c_ref[...] = bl_big * rs1_big + fcb_big


def _apply_kernel(x_ref, m_ref, c_ref, o_ref):
    y = jnp.dot(x_ref[...], m_ref[...],
                preferred_element_type=jnp.float32) + c_ref[...]
    o_ref[...] = y.astype(jnp.bfloat16)


def kernel(src, graph_edge_index, feature_edge_index, Wl, bl, Wr,
           conv1_w, conv2_w, fc_w, fc_b):
    bz, cap, seq, inv = src.shape
    B = bz * cap
    x2 = src.astype(jnp.bfloat16).reshape(B, F)
    fei = jnp.pad(feature_edge_index, ((0, 6), (0, 0)))
    w1p = jnp.transpose(conv1_w, (2, 1, 0))
    w2p = jnp.transpose(conv2_w, (2, 1, 0))

    M, c = pl.pallas_call(
        _fold_kernel,
        out_shape=(jax.ShapeDtypeStruct((F, F), jnp.float32),
                   jax.ShapeDtypeStruct((1, F), jnp.float32)),
    )(fei, Wl.T, Wr.T, bl.reshape(1, seq), w1p, w2p, fc_w.T,
      fc_b.reshape(1, inv))

    Bb = 8192
    out = pl.pallas_call(
        _apply_kernel,
        grid=(B // Bb,),
        in_specs=[pl.BlockSpec((Bb, F), lambda i: (i, 0)),
                  pl.BlockSpec((F, F), lambda i: (0, 0)),
                  pl.BlockSpec((1, F), lambda i: (0, 0))],
        out_specs=pl.BlockSpec((Bb, F), lambda i: (i, 0)),
        out_shape=jax.ShapeDtypeStruct((B, F), jnp.bfloat16),
    )(x2, M.astype(jnp.bfloat16), c)
    return out.reshape(bz, cap, seq, inv).astype(jnp.float32)
